# baseline (speedup 1.0000x reference)
import jax
import jax.numpy as jnp
from jax import lax
from jax.experimental import pallas as pl
from jax.experimental.pallas import tpu as pltpu
from jax.experimental.pallas import tpu_sc as plsc

N_NODES = 10000
N_EDGES = 320000
D_FEAT = 128
D_OUT = 128

NC = 2
NS = 16
NW = NC * NS

N_PAD = 10240
STRIPE = N_PAD // NS

CH = 128
NCHG = N_EDGES // CH


def _sc_body(src_hbm, dst_hbm, w_hbm, x_hbm, np_hbm, deg_hbm,
             src_ch, dst_ch, w_ch, rows, dtmp, acc_sh, deg_sh,
             gsem, isem, s1sem, s2sem):
  c = lax.axis_index("c")
  s = lax.axis_index("s")
  wid = s * NC + c
  n = NCHG // NW + jnp.where(wid < NCHG % NW, 1, 0)

  def _zrow(r, _):
    for j in range(D_FEAT // 16):
      rows[r, pl.ds(16 * j, 16)] = jnp.zeros((16,), jnp.float32)
    return 0
  lax.fori_loop(0, CH, _zrow, 0)
  for k in range(CH // 16):
    dtmp[pl.ds(16 * k, 16)] = jnp.zeros((16,), jnp.float32)
  for k in range(STRIPE // CH):
    r0 = s * STRIPE + k * CH
    pltpu.sync_copy(rows.at[pl.ds(0, CH)], acc_sh.at[pl.ds(r0, CH)])
    pltpu.sync_copy(dtmp, deg_sh.at[pl.ds(r0, CH)])
  plsc.subcore_barrier()

  def idx_start(j, q):
    base = (wid + NW * j) * CH
    pltpu.async_copy(src_hbm.at[pl.ds(base, CH)], src_ch.at[q], isem)
    pltpu.async_copy(dst_hbm.at[pl.ds(base, CH)], dst_ch.at[q], isem)
    pltpu.async_copy(w_hbm.at[pl.ds(base, CH)], w_ch.at[q], isem)

  def idx_wait(q):
    pltpu.make_async_copy(src_hbm.at[pl.ds(0, CH)], src_ch.at[q], isem).wait()
    pltpu.make_async_copy(dst_hbm.at[pl.ds(0, CH)], dst_ch.at[q], isem).wait()
    pltpu.make_async_copy(w_hbm.at[pl.ds(0, CH)], w_ch.at[q], isem).wait()

  def gather_start(q, p):
    pltpu.async_copy(x_hbm.at[src_ch.at[q]], rows.at[pl.ds(p * CH, CH)], gsem)

  def gather_wait(q, p):
    pltpu.make_async_copy(
        x_hbm.at[src_ch.at[q]], rows.at[pl.ds(p * CH, CH)], gsem).wait()

  def scatter_start(q, p):
    pltpu.async_copy(
        rows.at[pl.ds(p * CH, CH)], acc_sh.at[dst_ch.at[q]], s1sem, add=True)
    pltpu.async_copy(w_ch.at[q], deg_sh.at[dst_ch.at[q]], s2sem, add=True)

  def scatter_wait(q, p):
    pltpu.make_async_copy(
        rows.at[pl.ds(p * CH, CH)], acc_sh.at[dst_ch.at[q]], s1sem).wait()
    pltpu.make_async_copy(w_ch.at[q], deg_sh.at[dst_ch.at[q]], s2sem).wait()

  idx_start(0, 0)
  idx_wait(0)
  gather_start(0, 0)

  def _chunk(i, _):
    p = i % 2
    q = i % 3

    @pl.when(i + 1 < n)
    def _():
      idx_start(i + 1, (i + 1) % 3)

    gather_wait(q, p)

    def _scale(k, _):
      e0 = p * CH + 16 * k
      w16 = w_ch[q, pl.ds(16 * k, 16)]
      for l in range(16):
        wv = w16[l]
        for j in range(D_FEAT // 16):
          sl = pl.ds(16 * j, 16)
          rows[e0 + l, sl] = rows[e0 + l, sl] * wv
      return 0
    lax.fori_loop(0, CH // 16, _scale, 0)

    @pl.when(i + 1 < n)
    def _():
      @pl.when(i >= 1)
      def _():
        scatter_wait((i - 1) % 3, 1 - p)
      idx_wait((i + 1) % 3)
      gather_start((i + 1) % 3, 1 - p)

    scatter_start(q, p)
    return 0
  lax.fori_loop(0, n, _chunk, 0)
  scatter_wait((n - 1) % 3, (n - 1) % 2)
  scatter_wait((n - 2) % 3, n % 2)
  plsc.subcore_barrier()

  for k in range(STRIPE // CH):
    r0 = s * STRIPE + k * CH
    pltpu.sync_copy(acc_sh.at[pl.ds(r0, CH)], rows.at[pl.ds(0, CH)])
    pltpu.sync_copy(rows.at[pl.ds(0, CH)], np_hbm.at[c, pl.ds(r0, CH)])
    pltpu.sync_copy(deg_sh.at[pl.ds(r0, CH)], dtmp)
    pltpu.sync_copy(dtmp, deg_hbm.at[pl.ds(c * N_PAD + r0, CH)])


_sc_call = pl.kernel(
    _sc_body,
    out_type=(
        jax.ShapeDtypeStruct((NC, N_PAD, D_FEAT), jnp.float32),
        jax.ShapeDtypeStruct((NC * N_PAD,), jnp.float32),
    ),
    mesh=plsc.VectorSubcoreMesh(
        core_axis_name="c", subcore_axis_name="s", num_cores=NC,
        num_subcores=NS),
    scratch_types=(
        pltpu.VMEM((3, CH), jnp.int32),
        pltpu.VMEM((3, CH), jnp.int32),
        pltpu.VMEM((3, CH), jnp.float32),
        pltpu.VMEM((2 * CH, D_FEAT), jnp.float32),
        pltpu.VMEM((CH,), jnp.float32),
        pltpu.VMEM_SHARED((N_PAD, D_FEAT), jnp.float32),
        pltpu.VMEM_SHARED((N_PAD,), jnp.float32),
        pltpu.SemaphoreType.DMA,
        pltpu.SemaphoreType.DMA,
        pltpu.SemaphoreType.DMA,
        pltpu.SemaphoreType.DMA,
    ),
)


_TC_R = 1000


def _tc_body(x_ref, p0_ref, p1_ref, d0_ref, d1_ref, w1_ref, w2_ref, o_ref):
  d = d0_ref[...] + d1_ref[...]
  neigh = (p0_ref[...] + p1_ref[...]) / (d + 1e-9)
  acc = jnp.dot(x_ref[...], w1_ref[...], preferred_element_type=jnp.float32)
  acc = acc + jnp.dot(neigh, w2_ref[...], preferred_element_type=jnp.float32)
  o_ref[...] = acc * jax.nn.sigmoid(acc)


_tc_call = pl.pallas_call(
    _tc_body,
    grid=(N_NODES // _TC_R,),
    in_specs=[
        pl.BlockSpec((_TC_R, D_FEAT), lambda i: (i, 0)),
        pl.BlockSpec((_TC_R, D_FEAT), lambda i: (i, 0)),
        pl.BlockSpec((_TC_R, D_FEAT), lambda i: (i, 0)),
        pl.BlockSpec((_TC_R, 1), lambda i: (i, 0)),
        pl.BlockSpec((_TC_R, 1), lambda i: (i, 0)),
        pl.BlockSpec((D_FEAT, D_OUT), lambda i: (0, 0)),
        pl.BlockSpec((D_FEAT, D_OUT), lambda i: (0, 0)),
    ],
    out_specs=pl.BlockSpec((_TC_R, D_OUT), lambda i: (i, 0)),
    out_shape=jax.ShapeDtypeStruct((N_NODES, D_OUT), jnp.float32),
)


@jax.jit
def kernel(x, edge_index, edge_weight, W):
  src = edge_index[0].astype(jnp.int32)
  dst = edge_index[1].astype(jnp.int32)
  w = edge_weight.astype(jnp.float32)
  np_out, deg_out = _sc_call(src, dst, w, x)
  p0 = np_out[0, :N_NODES]
  p1 = np_out[1, :N_NODES]
  d0 = deg_out[:N_NODES].reshape(N_NODES, 1)
  d1 = deg_out[N_PAD:N_PAD + N_NODES].reshape(N_NODES, 1)
  return _tc_call(x, p0, p1, d0, d1, W[:D_FEAT], W[D_FEAT:])


# speedup vs baseline: 1.3901x; 1.3901x over previous
import jax
import jax.numpy as jnp
from jax import lax
from jax.experimental import pallas as pl
from jax.experimental.pallas import tpu as pltpu
from jax.experimental.pallas import tpu_sc as plsc

N_NODES = 10000
N_EDGES = 320000
D_FEAT = 128
D_OUT = 128

NC = 2
NS = 16
NW = NC * NS

N_PAD = 10240
STRIPE = N_PAD // NS

CH = 128
NCHG = N_EDGES // CH


def _sc_body(src_hbm, dst_hbm, w_hbm, x_hbm, np_hbm, deg_hbm,
             src_ch, dst_ch, w_ch, rows, dtmp, acc_sh, deg_sh,
             gsem, isem, s1sem, s2sem):
  c = lax.axis_index("c")
  s = lax.axis_index("s")
  wid = s * NC + c
  n = NCHG // NW + jnp.where(wid < NCHG % NW, 1, 0)

  def _zrow(r, _):
    for j in range(D_FEAT // 16):
      rows[r, pl.ds(16 * j, 16)] = jnp.zeros((16,), jnp.float32)
    return 0
  lax.fori_loop(0, CH, _zrow, 0)
  for k in range(CH // 16):
    dtmp[pl.ds(16 * k, 16)] = jnp.zeros((16,), jnp.float32)
  for k in range(STRIPE // CH):
    r0 = s * STRIPE + k * CH
    pltpu.sync_copy(rows.at[pl.ds(0, CH)], acc_sh.at[pl.ds(r0, CH)])
    pltpu.sync_copy(dtmp, deg_sh.at[pl.ds(r0, CH)])
  plsc.subcore_barrier()

  def idx_start(j, q):
    base = (wid + NW * j) * CH
    pltpu.async_copy(src_hbm.at[pl.ds(base, CH)], src_ch.at[q], isem)
    pltpu.async_copy(dst_hbm.at[pl.ds(base, CH)], dst_ch.at[q], isem)
    pltpu.async_copy(w_hbm.at[pl.ds(base, CH)], w_ch.at[q], isem)

  def idx_wait(q):
    pltpu.make_async_copy(src_hbm.at[pl.ds(0, CH)], src_ch.at[q], isem).wait()
    pltpu.make_async_copy(dst_hbm.at[pl.ds(0, CH)], dst_ch.at[q], isem).wait()
    pltpu.make_async_copy(w_hbm.at[pl.ds(0, CH)], w_ch.at[q], isem).wait()

  def gather_start(q, p):
    pltpu.async_copy(x_hbm.at[src_ch.at[q]], rows.at[pl.ds(p * CH, CH)], gsem)

  def gather_wait(q, p):
    pltpu.make_async_copy(
        x_hbm.at[src_ch.at[q]], rows.at[pl.ds(p * CH, CH)], gsem).wait()

  def scatter_start(q, p):
    pltpu.async_copy(
        rows.at[pl.ds(p * CH, CH)], acc_sh.at[dst_ch.at[q]], s1sem, add=True)
    pltpu.async_copy(w_ch.at[q], deg_sh.at[dst_ch.at[q]], s2sem, add=True)

  def scatter_wait(q, p):
    pltpu.make_async_copy(
        rows.at[pl.ds(p * CH, CH)], acc_sh.at[dst_ch.at[q]], s1sem).wait()
    pltpu.make_async_copy(w_ch.at[q], deg_sh.at[dst_ch.at[q]], s2sem).wait()

  def _chunk(i, _):
    base = (wid + NW * i) * CH
    pltpu.sync_copy(src_hbm.at[pl.ds(base, CH)], src_ch.at[0])
    pltpu.sync_copy(dst_hbm.at[pl.ds(base, CH)], dst_ch.at[0])
    pltpu.sync_copy(w_hbm.at[pl.ds(base, CH)], w_ch.at[0])
    pltpu.sync_copy(x_hbm.at[src_ch.at[0]], rows.at[pl.ds(0, CH)])

    def _scale(k, _):
      e0 = 16 * k
      w16 = w_ch[0, pl.ds(16 * k, 16)]
      for l in range(16):
        wv = w16[l]
        for j in range(D_FEAT // 16):
          sl = pl.ds(16 * j, 16)
          rows[e0 + l, sl] = rows[e0 + l, sl] * wv
      return 0
    lax.fori_loop(0, CH // 16, _scale, 0)

    pltpu.sync_copy(rows.at[pl.ds(0, CH)], acc_sh.at[dst_ch.at[0]], add=True)
    pltpu.sync_copy(w_ch.at[0], deg_sh.at[dst_ch.at[0]], add=True)
    return 0
  lax.fori_loop(0, n, _chunk, 0)
  plsc.subcore_barrier()

  for k in range(STRIPE // CH):
    r0 = s * STRIPE + k * CH
    pltpu.sync_copy(acc_sh.at[pl.ds(r0, CH)], rows.at[pl.ds(0, CH)])
    pltpu.sync_copy(rows.at[pl.ds(0, CH)], np_hbm.at[c, pl.ds(r0, CH)])
    pltpu.sync_copy(deg_sh.at[pl.ds(r0, CH)], dtmp)
    pltpu.sync_copy(dtmp, deg_hbm.at[pl.ds(c * N_PAD + r0, CH)])


_sc_call = pl.kernel(
    _sc_body,
    out_type=(
        jax.ShapeDtypeStruct((NC, N_PAD, D_FEAT), jnp.float32),
        jax.ShapeDtypeStruct((NC * N_PAD,), jnp.float32),
    ),
    mesh=plsc.VectorSubcoreMesh(
        core_axis_name="c", subcore_axis_name="s", num_cores=NC,
        num_subcores=NS),
    scratch_types=(
        pltpu.VMEM((3, CH), jnp.int32),
        pltpu.VMEM((3, CH), jnp.int32),
        pltpu.VMEM((3, CH), jnp.float32),
        pltpu.VMEM((2 * CH, D_FEAT), jnp.float32),
        pltpu.VMEM((CH,), jnp.float32),
        pltpu.VMEM_SHARED((N_PAD, D_FEAT), jnp.float32),
        pltpu.VMEM_SHARED((N_PAD,), jnp.float32),
        pltpu.SemaphoreType.DMA,
        pltpu.SemaphoreType.DMA,
        pltpu.SemaphoreType.DMA,
        pltpu.SemaphoreType.DMA,
    ),
)


_TC_R = 1000


def _tc_body(x_ref, p0_ref, p1_ref, d0_ref, d1_ref, w1_ref, w2_ref, o_ref):
  d = d0_ref[...] + d1_ref[...]
  neigh = (p0_ref[...] + p1_ref[...]) / (d + 1e-9)
  acc = jnp.dot(x_ref[...], w1_ref[...], preferred_element_type=jnp.float32)
  acc = acc + jnp.dot(neigh, w2_ref[...], preferred_element_type=jnp.float32)
  o_ref[...] = acc * jax.nn.sigmoid(acc)


_tc_call = pl.pallas_call(
    _tc_body,
    grid=(N_NODES // _TC_R,),
    in_specs=[
        pl.BlockSpec((_TC_R, D_FEAT), lambda i: (i, 0)),
        pl.BlockSpec((_TC_R, D_FEAT), lambda i: (i, 0)),
        pl.BlockSpec((_TC_R, D_FEAT), lambda i: (i, 0)),
        pl.BlockSpec((_TC_R, 1), lambda i: (i, 0)),
        pl.BlockSpec((_TC_R, 1), lambda i: (i, 0)),
        pl.BlockSpec((D_FEAT, D_OUT), lambda i: (0, 0)),
        pl.BlockSpec((D_FEAT, D_OUT), lambda i: (0, 0)),
    ],
    out_specs=pl.BlockSpec((_TC_R, D_OUT), lambda i: (i, 0)),
    out_shape=jax.ShapeDtypeStruct((N_NODES, D_OUT), jnp.float32),
)


@jax.jit
def kernel(x, edge_index, edge_weight, W):
  src = edge_index[0].astype(jnp.int32)
  dst = edge_index[1].astype(jnp.int32)
  w = edge_weight.astype(jnp.float32)
  np_out, deg_out = _sc_call(src, dst, w, x)
  p0 = np_out[0, :N_NODES]
  p1 = np_out[1, :N_NODES]
  d0 = deg_out[:N_NODES].reshape(N_NODES, 1)
  d1 = deg_out[N_PAD:N_PAD + N_NODES].reshape(N_NODES, 1)
  return _tc_call(x, p0, p1, d0, d1, W[:D_FEAT], W[D_FEAT:])


# speedup vs baseline: 2.6580x; 1.9121x over previous
import jax
import jax.numpy as jnp
from jax import lax
from jax.experimental import pallas as pl
from jax.experimental.pallas import tpu as pltpu
from jax.experimental.pallas import tpu_sc as plsc

N_NODES = 10000
N_EDGES = 320000
D_FEAT = 128
D_OUT = 128

NC = 2
NS = 16
NW = NC * NS

N_PAD = 10240
STRIPE = N_PAD // NS

EPW = N_EDGES // NW
CH = 80
NCH = EPW // CH


def _sc_body(src_hbm, dst_hbm, w_hbm, x_hbm, np_hbm, deg_hbm,
             src_all, w_all, src_ch, dst_ch, rows, dtmp,
             acc_sh, deg_sh, gsem0, gsem1, dsem0, dsem1, dsem2, dsem3,
             bsem0, bsem1, bsem2, bsem3):
  degsems = (dsem0, dsem1, dsem2, dsem3)
  bsems = (bsem0, bsem1, bsem2, bsem3)
  c = lax.axis_index("c")
  s = lax.axis_index("s")
  wid = s * NC + c

  def _zrow(r, _):
    for j in range(D_FEAT // 16):
      rows[r, pl.ds(16 * j, 16)] = jnp.zeros((16,), jnp.float32)
    return 0
  lax.fori_loop(0, CH, _zrow, 0)
  for k in range(CH // 16):
    dtmp[pl.ds(16 * k, 16)] = jnp.zeros((16,), jnp.float32)
  for k in range(STRIPE // CH):
    r0 = s * STRIPE + k * CH
    pltpu.sync_copy(rows.at[pl.ds(0, CH)], acc_sh.at[pl.ds(r0, CH)])
    pltpu.sync_copy(dtmp, deg_sh.at[pl.ds(r0, CH)])
  plsc.subcore_barrier()

  base = wid * EPW
  pltpu.sync_copy(src_hbm.at[pl.ds(base, EPW)], src_all)
  pltpu.sync_copy(w_hbm.at[pl.ds(base, EPW)], w_all)

  def idx_copy(ci, P):
    off = ci * CH
    for k in range(CH // 16):
      src_ch[P, pl.ds(16 * k, 16)] = src_all[pl.ds(off + 16 * k, 16)]
    pltpu.async_copy(
        dst_hbm.at[pl.ds(base + off, CH)], dst_ch.at[P], bsems[P])

  def dst_wait(P):
    pltpu.make_async_copy(
        dst_hbm.at[pl.ds(0, CH)], dst_ch.at[P], bsems[P]).wait()

  def gather_start(P, R, sem):
    pltpu.async_copy(x_hbm.at[src_ch.at[P]], rows.at[pl.ds(R * CH, CH)], sem)

  def gather_wait(P, R, sem):
    pltpu.make_async_copy(
        x_hbm.at[src_ch.at[P]], rows.at[pl.ds(R * CH, CH)], sem).wait()

  def deg_wait(P, sem):
    pltpu.make_async_copy(dtmp, deg_sh.at[dst_ch.at[P]], sem).wait()

  def do_chunk(ci, P, R, gsem_, dsem_):
    gather_wait(P, R, gsem_)

    def _scale(k, _):
      e0 = 16 * k
      w16 = w_all[pl.ds(ci * CH + e0, 16)]
      for l in range(16):
        wv = w16[l]
        for j in range(D_FEAT // 16):
          sl = pl.ds(16 * j, 16)
          rows[R * CH + e0 + l, sl] = rows[R * CH + e0 + l, sl] * wv
      return 0
    lax.fori_loop(0, CH // 16, _scale, 0)

    dst_wait(P)
    pltpu.sync_copy(
        rows.at[pl.ds(R * CH, CH)], acc_sh.at[dst_ch.at[P]], add=True)
    pltpu.async_copy(
        w_all.at[pl.ds(ci * CH, CH)], deg_sh.at[dst_ch.at[P]], dsem_,
        add=True)

  for P in range(4):
    for k in range(CH // 16):
      dst_ch[P, pl.ds(16 * k, 16)] = jnp.zeros((16,), jnp.int32)
  for P in (1, 2, 3):
    pltpu.async_copy(dtmp, deg_sh.at[dst_ch.at[P]], degsems[P], add=True)
  idx_copy(0, 0)
  gather_start(0, 0, gsem0)

  def _body(j, _):
    c0 = 4 * j
    deg_wait(1, degsems[1]); idx_copy(c0 + 1, 1); gather_start(1, 1, gsem1)
    do_chunk(c0, 0, 0, gsem0, degsems[0])
    deg_wait(2, degsems[2]); idx_copy(c0 + 2, 2); gather_start(2, 0, gsem0)
    do_chunk(c0 + 1, 1, 1, gsem1, degsems[1])
    deg_wait(3, degsems[3]); idx_copy(c0 + 3, 3); gather_start(3, 1, gsem1)
    do_chunk(c0 + 2, 2, 0, gsem0, degsems[2])
    deg_wait(0, degsems[0]); idx_copy(c0 + 4, 0); gather_start(0, 0, gsem0)
    do_chunk(c0 + 3, 3, 1, gsem1, degsems[3])
    return 0
  lax.fori_loop(0, (NCH - 1) // 4, _body, 0)
  do_chunk(NCH - 1, 0, 0, gsem0, degsems[0])
  for P in range(4):
    deg_wait(P, degsems[P])
  plsc.subcore_barrier()

  for k in range(STRIPE // CH):
    r0 = s * STRIPE + k * CH
    pltpu.sync_copy(acc_sh.at[pl.ds(r0, CH)], rows.at[pl.ds(0, CH)])
    pltpu.sync_copy(rows.at[pl.ds(0, CH)], np_hbm.at[c, pl.ds(r0, CH)])
    pltpu.sync_copy(deg_sh.at[pl.ds(r0, CH)], dtmp)
    pltpu.sync_copy(dtmp, deg_hbm.at[pl.ds(c * N_PAD + r0, CH)])


_sc_call = pl.kernel(
    _sc_body,
    out_type=(
        jax.ShapeDtypeStruct((NC, N_PAD, D_FEAT), jnp.float32),
        jax.ShapeDtypeStruct((NC * N_PAD,), jnp.float32),
    ),
    mesh=plsc.VectorSubcoreMesh(
        core_axis_name="c", subcore_axis_name="s", num_cores=NC,
        num_subcores=NS),
    scratch_types=(
        pltpu.VMEM((EPW,), jnp.int32),
        pltpu.VMEM((EPW,), jnp.float32),
        pltpu.VMEM((4, CH), jnp.int32),
        pltpu.VMEM((4, CH), jnp.int32),
        pltpu.VMEM((2 * CH, D_FEAT), jnp.float32),
        pltpu.VMEM((CH,), jnp.float32),
        pltpu.VMEM_SHARED((N_PAD, D_FEAT), jnp.float32),
        pltpu.VMEM_SHARED((N_PAD,), jnp.float32),
        pltpu.SemaphoreType.DMA,
        pltpu.SemaphoreType.DMA,
        pltpu.SemaphoreType.DMA,
        pltpu.SemaphoreType.DMA,
        pltpu.SemaphoreType.DMA,
        pltpu.SemaphoreType.DMA,
        pltpu.SemaphoreType.DMA,
        pltpu.SemaphoreType.DMA,
        pltpu.SemaphoreType.DMA,
        pltpu.SemaphoreType.DMA,
    ),
)


_TC_R = 1000


def _tc_body(x_ref, p0_ref, p1_ref, d0_ref, d1_ref, w1_ref, w2_ref, o_ref):
  d = d0_ref[...] + d1_ref[...]
  neigh = (p0_ref[...] + p1_ref[...]) / (d + 1e-9)
  acc = jnp.dot(x_ref[...], w1_ref[...], preferred_element_type=jnp.float32)
  acc = acc + jnp.dot(neigh, w2_ref[...], preferred_element_type=jnp.float32)
  o_ref[...] = acc * jax.nn.sigmoid(acc)


_tc_call = pl.pallas_call(
    _tc_body,
    grid=(N_NODES // _TC_R,),
    in_specs=[
        pl.BlockSpec((_TC_R, D_FEAT), lambda i: (i, 0)),
        pl.BlockSpec((_TC_R, D_FEAT), lambda i: (i, 0)),
        pl.BlockSpec((_TC_R, D_FEAT), lambda i: (i, 0)),
        pl.BlockSpec((_TC_R, 1), lambda i: (i, 0)),
        pl.BlockSpec((_TC_R, 1), lambda i: (i, 0)),
        pl.BlockSpec((D_FEAT, D_OUT), lambda i: (0, 0)),
        pl.BlockSpec((D_FEAT, D_OUT), lambda i: (0, 0)),
    ],
    out_specs=pl.BlockSpec((_TC_R, D_OUT), lambda i: (i, 0)),
    out_shape=jax.ShapeDtypeStruct((N_NODES, D_OUT), jnp.float32),
)


@jax.jit
def kernel(x, edge_index, edge_weight, W):
  src = edge_index[0].astype(jnp.int32)
  dst = edge_index[1].astype(jnp.int32)
  w = edge_weight.astype(jnp.float32)
  np_out, deg_out = _sc_call(src, dst, w, x)
  p0 = np_out[0, :N_NODES]
  p1 = np_out[1, :N_NODES]
  d0 = deg_out[:N_NODES].reshape(N_NODES, 1)
  d1 = deg_out[N_PAD:N_PAD + N_NODES].reshape(N_NODES, 1)
  return _tc_call(x, p0, p1, d0, d1, W[:D_FEAT], W[D_FEAT:])
